# baseline (speedup 1.0000x reference)
import jax
import jax.numpy as jnp
from jax import lax
from jax.experimental import pallas as pl
from jax.experimental.pallas import tpu as pltpu
from jax.experimental.pallas import tpu_sc as plsc

_B = 16384
_KD = 16
_NC = 2
_NS = 16
_NW = _NC * _NS
_BPW = _B // _NW
_CHUNK = 256
_NCHUNK = _BPW // _CHUNK
_NBLK = _CHUNK // 16


def _body(legs_hbm, votes_hbm, theta_hbm, beta_hbm, tmean_hbm, bmean_hbm,
          ov_hbm, out_hbm,
          legs_v, votes_v, tci_v, bci_v, trows_v, brows_v,
          tmean_v, bmean_v, ov_v, out_v, sem):
    wid = lax.axis_index("s") * _NC + lax.axis_index("c")
    base = wid * _BPW

    pltpu.sync_copy(legs_hbm.at[pl.ds(base, _BPW)], legs_v)
    pltpu.sync_copy(votes_hbm.at[pl.ds(base, _BPW)], votes_v)
    pltpu.sync_copy(ov_hbm, ov_v)

    cp_tm = pltpu.async_copy(tmean_hbm.at[legs_v], tmean_v, sem)
    cp_bm = pltpu.async_copy(bmean_hbm.at[votes_v], bmean_v, sem)

    for j in range(_BPW // 16):
        s = pl.ds(j * 16, 16)
        tci_v[s] = lax.shift_right_logical(legs_v[s], 3)
        bci_v[s] = lax.shift_right_logical(votes_v[s], 3)

    iota = lax.iota(jnp.int32, 16)
    cp_tm.wait()
    cp_bm.wait()
    ov = ov_v[...]

    for c in range(_NCHUNK):
        cs = pl.ds(c * _CHUNK, _CHUNK)
        cp_t = pltpu.async_copy(theta_hbm.at[tci_v.at[cs]], trows_v, sem)
        cp_b = pltpu.async_copy(beta_hbm.at[bci_v.at[cs]], brows_v, sem)
        cp_t.wait()
        cp_b.wait()
        for j in range(_NBLK):
            rbase = j * 16
            s = pl.ds(c * _CHUNK + rbase, 16)
            row_idx = rbase + iota
            tcol = lax.shift_left(jnp.bitwise_and(legs_v[s], 7), 4)
            bcol = lax.shift_left(jnp.bitwise_and(votes_v[s], 7), 4)
            acc = tmean_v[s] + bmean_v[s] + ov
            for k in range(_KD):
                t = plsc.load_gather(trows_v, [row_idx, tcol + k])
                b = plsc.load_gather(brows_v, [row_idx, bcol + k])
                acc = acc + t * b
            out_v[s] = acc

    pltpu.sync_copy(out_v, out_hbm.at[pl.ds(base, _BPW)])


def kernel(legs, votes, theta, beta, theta_mean, beta_mean, overall_mean):
    theta_w = theta.reshape(theta.shape[0] // 8, 128)
    beta_w = beta.reshape(beta.shape[0] // 8, 128)
    ov16 = jnp.broadcast_to(overall_mean, (16,))
    mesh = plsc.VectorSubcoreMesh(core_axis_name="c", subcore_axis_name="s")
    f = pl.kernel(
        _body,
        out_type=jax.ShapeDtypeStruct((_B,), jnp.float32),
        mesh=mesh,
        compiler_params=pltpu.CompilerParams(needs_layout_passes=False),
        scratch_types=[
            pltpu.VMEM((_BPW,), jnp.int32),
            pltpu.VMEM((_BPW,), jnp.int32),
            pltpu.VMEM((_BPW,), jnp.int32),
            pltpu.VMEM((_BPW,), jnp.int32),
            pltpu.VMEM((_CHUNK, 128), jnp.float32),
            pltpu.VMEM((_CHUNK, 128), jnp.float32),
            pltpu.VMEM((_BPW,), jnp.float32),
            pltpu.VMEM((_BPW,), jnp.float32),
            pltpu.VMEM((16,), jnp.float32),
            pltpu.VMEM((_BPW,), jnp.float32),
            pltpu.SemaphoreType.DMA,
        ],
    )
    return f(legs, votes, theta_w, beta_w, theta_mean, beta_mean, ov16)
